# baseline (speedup 1.0000x reference)
import functools

import jax
import jax.numpy as jnp
from jax import lax
from jax.experimental import pallas as pl
from jax.experimental.pallas import tpu as pltpu
from jax.experimental.pallas import tpu_sc as plsc

_K = 8192
_D = 256
_N = 8192
_BT = 1024
_KC = 512

_NC = 2
_NS = 16
_L = 16
_NW = _NC * _NS
_BPW = _N // _NW
_CH = 128
_NCH = _BPW // _CH
_KPC = _K // _NS


def _argmin_kernel(x_ref, e_ref, ind_ref, dsum_ref,
                   a_sc, bm_sc, bi_sc):
    t_id = pl.program_id(0)
    kc = pl.program_id(1)
    nkc = pl.num_programs(1)
    xf = x_ref[...]

    @pl.when(kc == 0)
    def _start():
        x = x_ref[...]
        a_sc[...] = jnp.sum(x * x, axis=1).reshape(1, _BT)
        bm_sc[...] = jnp.full((1, _BT), jnp.inf, dtype=jnp.float32)
        bi_sc[...] = jnp.zeros((1, _BT), dtype=jnp.int32)

    e = e_ref[pl.ds(kc * _KC, _KC), :]
    b = jax.lax.dot_general(
        e, xf, (((1,), (1,)), ((), ())),
        preferred_element_type=jnp.float32,
        precision=jax.lax.Precision.HIGHEST)
    c = jnp.sum(e * e, axis=1, keepdims=True)
    t = (a_sc[...] - 2.0 * b) + c
    m = jnp.min(t, axis=0, keepdims=True)
    io = jax.lax.broadcasted_iota(jnp.int32, (_KC, _BT), 0) + kc * _KC
    idxc = jnp.min(jnp.where(t == m, io, 2 ** 30), axis=0, keepdims=True)
    upd = m < bm_sc[...]
    bi_sc[...] = jnp.where(upd, idxc, bi_sc[...])
    bm_sc[...] = jnp.where(upd, m, bm_sc[...])

    @pl.when(kc == nkc - 1)
    def _finish():
        ind_ref[...] = bi_sc[...].reshape(1, 1, _BT)
        blk = jnp.sum(bm_sc[...])

        @pl.when(t_id == 0)
        def _init():
            dsum_ref[0, 0] = 0.0

        dsum_ref[0, 0] += blk


def _run_argmin(x2d, embed):
    ind, dsum = pl.pallas_call(
        _argmin_kernel,
        grid=(_N // _BT, _K // _KC),
        in_specs=[
            pl.BlockSpec((_BT, _D), lambda t, k: (t, 0)),
            pl.BlockSpec((_K, _D), lambda t, k: (0, 0)),
        ],
        out_specs=[
            pl.BlockSpec((1, 1, _BT), lambda t, k: (t, 0, 0)),
            pl.BlockSpec(memory_space=pltpu.SMEM, block_shape=(1, 1),
                         index_map=lambda t, k: (0, 0)),
        ],
        out_shape=[
            jax.ShapeDtypeStruct((_N // _BT, 1, _BT), jnp.int32),
            jax.ShapeDtypeStruct((1, 1), jnp.float32),
        ],
        scratch_shapes=[
            pltpu.VMEM((1, _BT), jnp.float32),
            pltpu.VMEM((1, _BT), jnp.float32),
            pltpu.VMEM((1, _BT), jnp.int32),
        ],
    )(x2d, embed)
    return ind.reshape(_NW * _NCH, _CH), dsum


@functools.cache
def _make_sc_gather_counts():
    mesh = plsc.VectorSubcoreMesh(core_axis_name="c", subcore_axis_name="s")
    return functools.partial(
        pl.kernel, mesh=mesh,
        out_type=[
            jax.ShapeDtypeStruct((_N, _D), jnp.float32),
            jax.ShapeDtypeStruct((_NC * _K,), jnp.float32),
        ],
        scratch_types=[
            pltpu.VMEM((_NCH, _CH), jnp.int32),
            pltpu.VMEM((_BPW, _D), jnp.float32),
            pltpu.VMEM((_CH,), jnp.float32),
            pltpu.VMEM((_KPC,), jnp.float32),
            pltpu.VMEM_SHARED((_K,), jnp.float32),
            pltpu.SemaphoreType.DMA,
        ],
    )(_sc_gather_counts_body)


def _sc_gather_counts_body(embed_hbm, idx_hbm, quant_hbm, cnt_hbm,
                           idx_v, rows_v, ones_v, slice_v, shared_cnt, sem):
    c = lax.axis_index("c")
    s = lax.axis_index("s")
    wid = s * _NC + c
    base = wid * _BPW

    pltpu.sync_copy(idx_hbm.at[pl.ds(wid * _NCH, _NCH)], idx_v)

    def _zero(i, _):
        slice_v[pl.ds(i * _L, _L)] = jnp.zeros((_L,), jnp.float32)
        return 0
    lax.fori_loop(0, _KPC // _L, _zero, 0)
    pltpu.sync_copy(slice_v, shared_cnt.at[pl.ds(s * _KPC, _KPC)])

    def _ones(i, _):
        ones_v[pl.ds(i * _L, _L)] = jnp.ones((_L,), jnp.float32)
        return 0
    lax.fori_loop(0, _CH // _L, _ones, 0)

    plsc.subcore_barrier()

    for j in range(_NCH):
        pltpu.async_copy(embed_hbm.at[idx_v.at[j]],
                         rows_v.at[pl.ds(j * _CH, _CH)], sem).wait()
        pltpu.sync_copy(ones_v, shared_cnt.at[idx_v.at[j]], add=True)

    pltpu.sync_copy(rows_v, quant_hbm.at[pl.ds(base, _BPW)])

    plsc.subcore_barrier()
    pltpu.sync_copy(shared_cnt.at[pl.ds(s * _KPC, _KPC)], slice_v)
    pltpu.sync_copy(slice_v, cnt_hbm.at[pl.ds(c * _K + s * _KPC, _KPC)])


def _finalize_kernel(cnt_ref, num_ref, perp_ref):
    cnts = jnp.sum(cnt_ref[...], axis=0, keepdims=True)
    num_ref[...] = cnts
    avg = cnts * (1.0 / _N)
    perp_ref[0, 0] = jnp.exp(-jnp.sum(avg * jnp.log(avg + 1e-10)))


def _run_finalize(cnt2):
    num, perp = pl.pallas_call(
        _finalize_kernel,
        in_specs=[pl.BlockSpec((_NC, _K), lambda: (0, 0))],
        out_specs=[
            pl.BlockSpec((1, _K), lambda: (0, 0)),
            pl.BlockSpec(memory_space=pltpu.SMEM, block_shape=(1, 1),
                         index_map=lambda: (0, 0)),
        ],
        out_shape=[
            jax.ShapeDtypeStruct((1, _K), jnp.float32),
            jax.ShapeDtypeStruct((1, 1), jnp.float32),
        ],
    )(cnt2)
    return num.reshape(_K), perp[0, 0]


def kernel(input, embed):
    shape = input.shape
    x2d = input.reshape(-1, shape[-1])
    idx64, dsum = _run_argmin(x2d, embed)
    commit_loss = dsum[0, 0] / jnp.float32(_N * _D)
    quant2d, cnt2 = _make_sc_gather_counts()(embed, idx64)
    embed_num, perplexity = _run_finalize(cnt2.reshape(_NC, _K))
    quantize = quant2d.reshape(shape)
    return (quantize, embed_num, commit_loss, perplexity)


# speedup vs baseline: 2.1485x; 2.1485x over previous
import functools

import jax
import jax.numpy as jnp
from jax import lax
from jax.experimental import pallas as pl
from jax.experimental.pallas import tpu as pltpu
from jax.experimental.pallas import tpu_sc as plsc

_K = 8192
_D = 256
_N = 8192
_BT = 1024
_KC = 512

_NC = 2
_NS = 16
_L = 16
_NW = _NC * _NS
_BPW = _N // _NW
_CH = 128
_NCH = _BPW // _CH
_KPC = _K // _NS


def _argmin_kernel(x_ref, e_ref, ind_ref, dsum_ref,
                   a_sc, bm_sc, bi_sc):
    t_id = pl.program_id(0)
    kc = pl.program_id(1)
    nkc = pl.num_programs(1)
    xf = x_ref[...]

    @pl.when(kc == 0)
    def _start():
        x = x_ref[...]
        a_sc[...] = jnp.sum(x * x, axis=1).reshape(1, _BT)
        bm_sc[...] = jnp.full((1, _BT), jnp.inf, dtype=jnp.float32)
        bi_sc[...] = jnp.zeros((1, _BT), dtype=jnp.int32)

    e = e_ref[pl.ds(kc * _KC, _KC), :]
    b = jax.lax.dot_general(
        e, xf, (((1,), (1,)), ((), ())),
        preferred_element_type=jnp.float32)
    c = jnp.sum(e * e, axis=1, keepdims=True)
    t = (a_sc[...] - 2.0 * b) + c
    m = jnp.min(t, axis=0, keepdims=True)
    io = jax.lax.broadcasted_iota(jnp.int32, (_KC, _BT), 0) + kc * _KC
    idxc = jnp.min(jnp.where(t == m, io, 2 ** 30), axis=0, keepdims=True)
    upd = m < bm_sc[...]
    bi_sc[...] = jnp.where(upd, idxc, bi_sc[...])
    bm_sc[...] = jnp.where(upd, m, bm_sc[...])

    @pl.when(kc == nkc - 1)
    def _finish():
        ind_ref[...] = bi_sc[...].reshape(1, 1, _BT)
        blk = jnp.sum(bm_sc[...])

        @pl.when(t_id == 0)
        def _init():
            dsum_ref[0, 0] = 0.0

        dsum_ref[0, 0] += blk


def _run_argmin(x2d, embed):
    ind, dsum = pl.pallas_call(
        _argmin_kernel,
        grid=(_N // _BT, _K // _KC),
        in_specs=[
            pl.BlockSpec((_BT, _D), lambda t, k: (t, 0)),
            pl.BlockSpec((_K, _D), lambda t, k: (0, 0)),
        ],
        out_specs=[
            pl.BlockSpec((1, 1, _BT), lambda t, k: (t, 0, 0)),
            pl.BlockSpec(memory_space=pltpu.SMEM, block_shape=(1, 1),
                         index_map=lambda t, k: (0, 0)),
        ],
        out_shape=[
            jax.ShapeDtypeStruct((_N // _BT, 1, _BT), jnp.int32),
            jax.ShapeDtypeStruct((1, 1), jnp.float32),
        ],
        scratch_shapes=[
            pltpu.VMEM((1, _BT), jnp.float32),
            pltpu.VMEM((1, _BT), jnp.float32),
            pltpu.VMEM((1, _BT), jnp.int32),
        ],
    )(x2d, embed)
    return ind.reshape(_NW * _NCH, _CH), dsum


@functools.cache
def _make_sc_gather_counts():
    mesh = plsc.VectorSubcoreMesh(core_axis_name="c", subcore_axis_name="s")
    return functools.partial(
        pl.kernel, mesh=mesh,
        out_type=[
            jax.ShapeDtypeStruct((_N, _D), jnp.float32),
            jax.ShapeDtypeStruct((_NC * _K,), jnp.float32),
        ],
        scratch_types=[
            pltpu.VMEM((_NCH, _CH), jnp.int32),
            pltpu.VMEM((_BPW, _D), jnp.float32),
            pltpu.VMEM((_CH,), jnp.float32),
            pltpu.VMEM((_KPC,), jnp.float32),
            pltpu.VMEM_SHARED((_K,), jnp.float32),
            pltpu.SemaphoreType.DMA,
        ],
    )(_sc_gather_counts_body)


def _sc_gather_counts_body(embed_hbm, idx_hbm, quant_hbm, cnt_hbm,
                           idx_v, rows_v, ones_v, slice_v, shared_cnt, sem):
    c = lax.axis_index("c")
    s = lax.axis_index("s")
    wid = s * _NC + c
    base = wid * _BPW

    pltpu.sync_copy(idx_hbm.at[pl.ds(wid * _NCH, _NCH)], idx_v)

    def _zero(i, _):
        slice_v[pl.ds(i * _L, _L)] = jnp.zeros((_L,), jnp.float32)
        return 0
    lax.fori_loop(0, _KPC // _L, _zero, 0)
    pltpu.sync_copy(slice_v, shared_cnt.at[pl.ds(s * _KPC, _KPC)])

    def _ones(i, _):
        ones_v[pl.ds(i * _L, _L)] = jnp.ones((_L,), jnp.float32)
        return 0
    lax.fori_loop(0, _CH // _L, _ones, 0)

    plsc.subcore_barrier()

    for j in range(_NCH):
        pltpu.async_copy(embed_hbm.at[idx_v.at[j]],
                         rows_v.at[pl.ds(j * _CH, _CH)], sem).wait()
        pltpu.sync_copy(ones_v, shared_cnt.at[idx_v.at[j]], add=True)

    pltpu.sync_copy(rows_v, quant_hbm.at[pl.ds(base, _BPW)])

    plsc.subcore_barrier()
    pltpu.sync_copy(shared_cnt.at[pl.ds(s * _KPC, _KPC)], slice_v)
    pltpu.sync_copy(slice_v, cnt_hbm.at[pl.ds(c * _K + s * _KPC, _KPC)])


def _finalize_kernel(cnt_ref, num_ref, perp_ref):
    cnts = jnp.sum(cnt_ref[...], axis=0, keepdims=True)
    num_ref[...] = cnts
    avg = cnts * (1.0 / _N)
    perp_ref[0, 0] = jnp.exp(-jnp.sum(avg * jnp.log(avg + 1e-10)))


def _run_finalize(cnt2):
    num, perp = pl.pallas_call(
        _finalize_kernel,
        in_specs=[pl.BlockSpec((_NC, _K), lambda: (0, 0))],
        out_specs=[
            pl.BlockSpec((1, _K), lambda: (0, 0)),
            pl.BlockSpec(memory_space=pltpu.SMEM, block_shape=(1, 1),
                         index_map=lambda: (0, 0)),
        ],
        out_shape=[
            jax.ShapeDtypeStruct((1, _K), jnp.float32),
            jax.ShapeDtypeStruct((1, 1), jnp.float32),
        ],
    )(cnt2)
    return num.reshape(_K), perp[0, 0]


def kernel(input, embed):
    shape = input.shape
    x2d = input.reshape(-1, shape[-1])
    idx64, dsum = _run_argmin(x2d, embed)
    commit_loss = dsum[0, 0] / jnp.float32(_N * _D)
    quant2d, cnt2 = _make_sc_gather_counts()(embed, idx64)
    embed_num, perplexity = _run_finalize(cnt2.reshape(_NC, _K))
    quantize = quant2d.reshape(shape)
    return (quantize, embed_num, commit_loss, perplexity)


# speedup vs baseline: 2.1743x; 1.0120x over previous
import functools

import jax
import jax.numpy as jnp
from jax import lax
from jax.experimental import pallas as pl
from jax.experimental.pallas import tpu as pltpu
from jax.experimental.pallas import tpu_sc as plsc

_K = 8192
_D = 256
_N = 8192
_BT = 1024
_KC = 512

_NC = 2
_NS = 16
_L = 16
_NW = _NC * _NS
_BPW = _N // _NW
_CH = 128
_NCH = _BPW // _CH
_KPC = _K // _NS


def _argmin_kernel(x_ref, e_ref, ind_ref, dsum_ref,
                   a_sc, bm_sc, bi_sc):
    t_id = pl.program_id(0)
    kc = pl.program_id(1)
    nkc = pl.num_programs(1)
    xf = x_ref[...]

    @pl.when(kc == 0)
    def _start():
        x = x_ref[...]
        a_sc[...] = jnp.sum(x * x, axis=1).reshape(1, _BT)
        bm_sc[...] = jnp.full((1, _BT), jnp.inf, dtype=jnp.float32)
        bi_sc[...] = jnp.zeros((1, _BT), dtype=jnp.int32)

    e = e_ref[pl.ds(kc * _KC, _KC), :]
    b2 = jax.lax.dot_general(
        e, xf + xf, (((1,), (1,)), ((), ())),
        preferred_element_type=jnp.float32)
    c = jnp.sum(e * e, axis=1, keepdims=True)
    t = (a_sc[...] - b2) + c
    m = jnp.min(t, axis=0, keepdims=True)
    io = jax.lax.broadcasted_iota(jnp.int32, (_KC, _BT), 0) + kc * _KC
    idxc = jnp.min(jnp.where(t == m, io, 2 ** 30), axis=0, keepdims=True)
    upd = m < bm_sc[...]
    bi_sc[...] = jnp.where(upd, idxc, bi_sc[...])
    bm_sc[...] = jnp.where(upd, m, bm_sc[...])

    @pl.when(kc == nkc - 1)
    def _finish():
        ind_ref[...] = bi_sc[...].reshape(1, 1, _BT)
        blk = jnp.sum(bm_sc[...])

        @pl.when(t_id == 0)
        def _init():
            dsum_ref[0, 0] = 0.0

        dsum_ref[0, 0] += blk


def _run_argmin(x2d, embed):
    ind, dsum = pl.pallas_call(
        _argmin_kernel,
        grid=(_N // _BT, _K // _KC),
        in_specs=[
            pl.BlockSpec((_BT, _D), lambda t, k: (t, 0)),
            pl.BlockSpec((_K, _D), lambda t, k: (0, 0)),
        ],
        out_specs=[
            pl.BlockSpec((1, 1, _BT), lambda t, k: (t, 0, 0)),
            pl.BlockSpec(memory_space=pltpu.SMEM, block_shape=(1, 1),
                         index_map=lambda t, k: (0, 0)),
        ],
        out_shape=[
            jax.ShapeDtypeStruct((_N // _BT, 1, _BT), jnp.int32),
            jax.ShapeDtypeStruct((1, 1), jnp.float32),
        ],
        scratch_shapes=[
            pltpu.VMEM((1, _BT), jnp.float32),
            pltpu.VMEM((1, _BT), jnp.float32),
            pltpu.VMEM((1, _BT), jnp.int32),
        ],
    )(x2d, embed)
    return ind.reshape(_NW * _NCH, _CH), dsum


@functools.cache
def _make_sc_gather_counts():
    mesh = plsc.VectorSubcoreMesh(core_axis_name="c", subcore_axis_name="s")
    return functools.partial(
        pl.kernel, mesh=mesh,
        out_type=[
            jax.ShapeDtypeStruct((_N, _D), jnp.float32),
            jax.ShapeDtypeStruct((_NC * _K,), jnp.float32),
        ],
        scratch_types=[
            pltpu.VMEM((_NCH, _CH), jnp.int32),
            pltpu.VMEM((_BPW, _D), jnp.float32),
            pltpu.VMEM((_CH,), jnp.float32),
            pltpu.VMEM((_KPC,), jnp.float32),
            pltpu.VMEM_SHARED((_K,), jnp.float32),
            pltpu.SemaphoreType.DMA,
        ],
    )(_sc_gather_counts_body)


def _sc_gather_counts_body(embed_hbm, idx_hbm, quant_hbm, cnt_hbm,
                           idx_v, rows_v, ones_v, slice_v, shared_cnt, sem):
    c = lax.axis_index("c")
    s = lax.axis_index("s")
    wid = s * _NC + c
    base = wid * _BPW

    pltpu.sync_copy(idx_hbm.at[pl.ds(wid * _NCH, _NCH)], idx_v)

    def _zero(i, _):
        slice_v[pl.ds(i * _L, _L)] = jnp.zeros((_L,), jnp.float32)
        return 0
    lax.fori_loop(0, _KPC // _L, _zero, 0)
    pltpu.sync_copy(slice_v, shared_cnt.at[pl.ds(s * _KPC, _KPC)])

    def _ones(i, _):
        ones_v[pl.ds(i * _L, _L)] = jnp.ones((_L,), jnp.float32)
        return 0
    lax.fori_loop(0, _CH // _L, _ones, 0)

    plsc.subcore_barrier()

    for j in range(_NCH):
        pltpu.async_copy(embed_hbm.at[idx_v.at[j]],
                         rows_v.at[pl.ds(j * _CH, _CH)], sem).wait()
        pltpu.sync_copy(ones_v, shared_cnt.at[idx_v.at[j]], add=True)

    pltpu.sync_copy(rows_v, quant_hbm.at[pl.ds(base, _BPW)])

    plsc.subcore_barrier()
    pltpu.sync_copy(shared_cnt.at[pl.ds(s * _KPC, _KPC)], slice_v)
    pltpu.sync_copy(slice_v, cnt_hbm.at[pl.ds(c * _K + s * _KPC, _KPC)])


def _finalize_kernel(cnt_ref, num_ref, perp_ref):
    cnts = jnp.sum(cnt_ref[...], axis=0, keepdims=True)
    num_ref[...] = cnts
    avg = cnts * (1.0 / _N)
    perp_ref[0, 0] = jnp.exp(-jnp.sum(avg * jnp.log(avg + 1e-10)))


def _run_finalize(cnt2):
    num, perp = pl.pallas_call(
        _finalize_kernel,
        in_specs=[pl.BlockSpec((_NC, _K), lambda: (0, 0))],
        out_specs=[
            pl.BlockSpec((1, _K), lambda: (0, 0)),
            pl.BlockSpec(memory_space=pltpu.SMEM, block_shape=(1, 1),
                         index_map=lambda: (0, 0)),
        ],
        out_shape=[
            jax.ShapeDtypeStruct((1, _K), jnp.float32),
            jax.ShapeDtypeStruct((1, 1), jnp.float32),
        ],
    )(cnt2)
    return num.reshape(_K), perp[0, 0]


def kernel(input, embed):
    shape = input.shape
    x2d = input.reshape(-1, shape[-1])
    idx64, dsum = _run_argmin(x2d, embed)
    commit_loss = dsum[0, 0] / jnp.float32(_N * _D)
    quant2d, cnt2 = _make_sc_gather_counts()(embed, idx64)
    embed_num, perplexity = _run_finalize(cnt2.reshape(_NC, _K))
    quantize = quant2d.reshape(shape)
    return (quantize, embed_num, commit_loss, perplexity)
